# baseline (speedup 1.0000x reference)
import functools

import jax
import jax.numpy as jnp
from jax import lax
from jax.experimental import pallas as pl
from jax.experimental.pallas import tpu as pltpu
from jax.experimental.pallas import tpu_sc as plsc

T = 26
VOCAB = 100000
D = 32
B = 4096
L = 20

NC = 2
NS = 16
NW = NC * NS

C = 64
ROWS = C * L
IDXW = 128
NDMA = ROWS // IDXW

BC = 32768
QB = BC // 4
SH = QB.bit_length() - 1
PIECES = ((0, 5), (5, 10), (10, 15), (15, 20), (20, 25), (25, 26))


def _relayout_body(w_ref, o_ref):
    x = w_ref[...]
    x4 = jnp.concatenate(
        [x[:, k * QB: (k + 1) * QB] for k in range(4)], axis=0
    )
    d_iota = lax.broadcasted_iota(jnp.int32, (128, 128), 0)
    l_iota = lax.broadcasted_iota(jnp.int32, (128, 128), 1)
    eye = (l_iota == d_iota).astype(jnp.float32)
    o_ref[...] = lax.dot_general(
        x4, eye, (((0,), (0,)), ((), ())),
        preferred_element_type=jnp.float32,
    )


def _relayout_piece(w_t, t0, t1):
    b0 = (t0 * VOCAB) // BC
    b1 = -(-(t1 * VOCAB) // BC)
    nblk = b1 - b0
    wide = pl.pallas_call(
        _relayout_body,
        grid=(nblk,),
        in_specs=[pl.BlockSpec((D, BC), lambda i, _b0=b0: (0, i + _b0))],
        out_specs=pl.BlockSpec((BC // 4, 128), lambda i: (i, 0)),
        out_shape=jax.ShapeDtypeStruct((nblk * BC * D // 128, 128), jnp.float32),
    )(w_t)
    return wide.reshape(nblk * BC, D), b0 * BC


def _gather_piece(indices, scratch, row_off, t0, t1):
    tp = t1 - t0
    bpw = tp * B // NW
    chunks = bpw // C
    srows = scratch.shape[0]

    mesh = plsc.VectorSubcoreMesh(core_axis_name="c", subcore_axis_name="s")

    @functools.partial(
        pl.kernel,
        out_type=jax.ShapeDtypeStruct((B * tp, D), jnp.float32),
        mesh=mesh,
        compiler_params=pltpu.CompilerParams(use_tc_tiling_on_sc=False),
        name=f"tbe_gather_pool_t{t0}_{t1}",
        scratch_types=[
            pltpu.VMEM((ROWS,), jnp.int32),
            pltpu.VMEM((ROWS,), jnp.int32),
            pltpu.VMEM((NDMA, IDXW), jnp.int32),
            pltpu.VMEM((NDMA, IDXW), jnp.int32),
            pltpu.VMEM((ROWS, D), jnp.float32),
            pltpu.VMEM((ROWS, D), jnp.float32),
            pltpu.VMEM((C, D), jnp.float32),
            pltpu.VMEM((C, D), jnp.float32),
            pltpu.VMEM((C,), jnp.int32),
            pltpu.VMEM((C,), jnp.int32),
            pltpu.SemaphoreType.DMA,
            pltpu.SemaphoreType.DMA,
            pltpu.SemaphoreType.DMA,
            pltpu.SemaphoreType.DMA,
        ],
    )
    def run(idx_hbm, w_hbm, out_hbm,
            raw0, raw1, gidx0, gidx1, rows0, rows1, pool0, pool1,
            dst0, dst1, sg0, sg1, so0, so1):
        wid = lax.axis_index("s") * NC + lax.axis_index("c")
        lanes = lax.iota(jnp.int32, 16)
        bufs = (
            (raw0, gidx0, rows0, sg0, pool0, dst0, so0),
            (raw1, gidx1, rows1, sg1, pool1, dst1, so1),
        )

        def fire(k, b):
            raw, gidx, rows, sg = bufs[b][:4]
            bag0 = t0 * B + wid * bpw + k * C
            t = bag0 // B
            pltpu.sync_copy(idx_hbm.at[pl.ds(bag0 * L, ROWS)], raw)
            t_off = t * VOCAB - row_off
            for i in range(ROWS // 16):
                g = raw[pl.ds(i * 16, 16)] + t_off
                v = (g & ~(BC - 1)) | ((g & (QB - 1)) << 2) | ((g >> SH) & 3)
                gidx[i // (IDXW // 16), pl.ds((i % (IDXW // 16)) * 16, 16)] = v
            for j in range(NDMA):
                pltpu.async_copy(
                    w_hbm.at[gidx.at[j]], rows.at[pl.ds(j * IDXW, IDXW)], sg
                )

        def drain_gathers(b):
            rows, sg = bufs[b][2:4]
            pltpu.make_async_copy(w_hbm.at[pl.ds(0, ROWS)], rows, sg).wait()

        def pool_scatter(k, b):
            rows = bufs[b][2]
            pool, dstv, so = bufs[b][4:7]
            bag0 = t0 * B + wid * bpw + k * C
            t = bag0 // B
            bb0 = bag0 - t * B

            def pool_body(c, carry2):
                r0 = c * L
                a0 = rows[r0, pl.ds(0, 16)]
                a1 = rows[r0, pl.ds(16, 16)]
                for l in range(1, L):
                    a0 = a0 + rows[r0 + l, pl.ds(0, 16)]
                    a1 = a1 + rows[r0 + l, pl.ds(16, 16)]
                pool[c, pl.ds(0, 16)] = a0
                pool[c, pl.ds(16, 16)] = a1
                return carry2

            lax.fori_loop(0, C, pool_body, 0)
            for q in range(C // 16):
                dstv[pl.ds(q * 16, 16)] = (bb0 + q * 16 + lanes) * tp + (t - t0)
            pltpu.async_copy(pool, out_hbm.at[dstv], so)

        def drain_scatter(b):
            pool, _, so = bufs[b][4:7]
            pltpu.make_async_copy(pool, out_hbm.at[pl.ds(0, C)], so).wait()

        fire(0, 0)

        def body(kk, carry):
            e = kk * 2
            fire(e + 1, 1)
            drain_gathers(0)
            pool_scatter(e, 0)
            fire(e + 2, 0)
            drain_gathers(1)
            pool_scatter(e + 1, 1)
            drain_scatter(0)
            drain_scatter(1)
            return carry

        lax.fori_loop(0, chunks // 2 - 1, body, 0)
        e = chunks - 2
        fire(e + 1, 1)
        drain_gathers(0)
        pool_scatter(e, 0)
        drain_gathers(1)
        pool_scatter(e + 1, 1)
        drain_scatter(0)
        drain_scatter(1)

    del srows
    return run(indices, scratch)


def kernel(indices, offsets, weights):
    del offsets
    w_t = weights.T
    parts = []
    for t0, t1 in PIECES:
        scratch, row_off = _relayout_piece(w_t, t0, t1)
        out_p = _gather_piece(indices, scratch, row_off, t0, t1)
        parts.append(out_p.reshape(B, (t1 - t0) * D))
    return jnp.concatenate(parts, axis=1)


# speedup vs baseline: 1.0527x; 1.0527x over previous
import functools

import jax
import jax.numpy as jnp
from jax import lax
from jax.experimental import pallas as pl
from jax.experimental.pallas import tpu as pltpu
from jax.experimental.pallas import tpu_sc as plsc

T = 26
VOCAB = 100000
D = 32
B = 4096
L = 20

NC = 2
NS = 16
NW = NC * NS

C = 64
ROWS = C * L
IDXW = 128
NDMA = ROWS // IDXW

BC = 32768
QB = BC // 4
SH = QB.bit_length() - 1
PIECES = ((0, 8), (8, 16), (16, 24), (24, 26))


def _relayout_body(w_ref, o_ref):
    x = w_ref[...]
    x4 = jnp.concatenate(
        [x[:, k * QB: (k + 1) * QB] for k in range(4)], axis=0
    )
    d_iota = lax.broadcasted_iota(jnp.int32, (128, 128), 0)
    l_iota = lax.broadcasted_iota(jnp.int32, (128, 128), 1)
    eye = (l_iota == d_iota).astype(jnp.float32)
    o_ref[...] = lax.dot_general(
        x4, eye, (((0,), (0,)), ((), ())),
        preferred_element_type=jnp.float32,
    )


def _relayout_piece(w_t, t0, t1):
    b0 = (t0 * VOCAB) // BC
    b1 = -(-(t1 * VOCAB) // BC)
    nblk = b1 - b0
    wide = pl.pallas_call(
        _relayout_body,
        grid=(nblk,),
        in_specs=[pl.BlockSpec((D, BC), lambda i, _b0=b0: (0, i + _b0))],
        out_specs=pl.BlockSpec((BC // 4, 128), lambda i: (i, 0)),
        out_shape=jax.ShapeDtypeStruct((nblk * BC * D // 128, 128), jnp.float32),
    )(w_t)
    return wide.reshape(nblk * BC, D), b0 * BC


def _gather_piece(indices, scratch, row_off, t0, t1):
    tp = t1 - t0
    bpw = tp * B // NW
    chunks = bpw // C
    srows = scratch.shape[0]

    mesh = plsc.VectorSubcoreMesh(core_axis_name="c", subcore_axis_name="s")

    @functools.partial(
        pl.kernel,
        out_type=jax.ShapeDtypeStruct((B * tp, D), jnp.float32),
        mesh=mesh,
        compiler_params=pltpu.CompilerParams(use_tc_tiling_on_sc=False),
        name=f"tbe_gather_pool_t{t0}_{t1}",
        scratch_types=[
            pltpu.VMEM((ROWS,), jnp.int32),
            pltpu.VMEM((ROWS,), jnp.int32),
            pltpu.VMEM((NDMA, IDXW), jnp.int32),
            pltpu.VMEM((NDMA, IDXW), jnp.int32),
            pltpu.VMEM((ROWS, D), jnp.float32),
            pltpu.VMEM((ROWS, D), jnp.float32),
            pltpu.VMEM((C, D), jnp.float32),
            pltpu.VMEM((C, D), jnp.float32),
            pltpu.VMEM((C,), jnp.int32),
            pltpu.VMEM((C,), jnp.int32),
            pltpu.SemaphoreType.DMA,
            pltpu.SemaphoreType.DMA,
            pltpu.SemaphoreType.DMA,
            pltpu.SemaphoreType.DMA,
        ],
    )
    def run(idx_hbm, w_hbm, out_hbm,
            raw0, raw1, gidx0, gidx1, rows0, rows1, pool0, pool1,
            dst0, dst1, sg0, sg1, so0, so1):
        wid = lax.axis_index("s") * NC + lax.axis_index("c")
        lanes = lax.iota(jnp.int32, 16)
        bufs = (
            (raw0, gidx0, rows0, sg0, pool0, dst0, so0),
            (raw1, gidx1, rows1, sg1, pool1, dst1, so1),
        )

        def fire(k, b):
            raw, gidx, rows, sg = bufs[b][:4]
            bag0 = t0 * B + wid * bpw + k * C
            t = bag0 // B
            pltpu.sync_copy(idx_hbm.at[pl.ds(bag0 * L, ROWS)], raw)
            t_off = t * VOCAB - row_off
            for i in range(ROWS // 16):
                g = raw[pl.ds(i * 16, 16)] + t_off
                v = (g & ~(BC - 1)) | ((g & (QB - 1)) << 2) | ((g >> SH) & 3)
                gidx[i // (IDXW // 16), pl.ds((i % (IDXW // 16)) * 16, 16)] = v
            for j in range(NDMA):
                pltpu.async_copy(
                    w_hbm.at[gidx.at[j]], rows.at[pl.ds(j * IDXW, IDXW)], sg
                )

        def drain_gathers(b):
            rows, sg = bufs[b][2:4]
            pltpu.make_async_copy(w_hbm.at[pl.ds(0, ROWS)], rows, sg).wait()

        def pool_scatter(k, b):
            rows = bufs[b][2]
            pool, dstv, so = bufs[b][4:7]
            bag0 = t0 * B + wid * bpw + k * C
            t = bag0 // B
            bb0 = bag0 - t * B

            def pool_body(c, carry2):
                r0 = c * L
                a0 = rows[r0, pl.ds(0, 16)]
                a1 = rows[r0, pl.ds(16, 16)]
                for l in range(1, L):
                    a0 = a0 + rows[r0 + l, pl.ds(0, 16)]
                    a1 = a1 + rows[r0 + l, pl.ds(16, 16)]
                pool[c, pl.ds(0, 16)] = a0
                pool[c, pl.ds(16, 16)] = a1
                return carry2

            lax.fori_loop(0, C, pool_body, 0)
            for q in range(C // 16):
                dstv[pl.ds(q * 16, 16)] = (bb0 + q * 16 + lanes) * tp + (t - t0)
            pltpu.async_copy(pool, out_hbm.at[dstv], so)

        def drain_scatter(b):
            pool, _, so = bufs[b][4:7]
            pltpu.make_async_copy(pool, out_hbm.at[pl.ds(0, C)], so).wait()

        fire(0, 0)

        def body(kk, carry):
            e = kk * 2
            fire(e + 1, 1)
            drain_gathers(0)
            pool_scatter(e, 0)
            fire(e + 2, 0)
            drain_gathers(1)
            pool_scatter(e + 1, 1)
            drain_scatter(0)
            drain_scatter(1)
            return carry

        lax.fori_loop(0, chunks // 2 - 1, body, 0)
        e = chunks - 2
        fire(e + 1, 1)
        drain_gathers(0)
        pool_scatter(e, 0)
        drain_gathers(1)
        pool_scatter(e + 1, 1)
        drain_scatter(0)
        drain_scatter(1)

    del srows
    return run(indices, scratch)


def kernel(indices, offsets, weights):
    del offsets
    w_t = weights.T
    parts = []
    for t0, t1 in PIECES:
        scratch, row_off = _relayout_piece(w_t, t0, t1)
        out_p = _gather_piece(indices, scratch, row_off, t0, t1)
        parts.append(out_p.reshape(B, (t1 - t0) * D))
    return jnp.concatenate(parts, axis=1)
